# baseline (speedup 1.0000x reference)
import jax
import jax.numpy as jnp
from jax import lax
from jax.experimental import pallas as pl
from jax.experimental.pallas import tpu as pltpu

VOCAB = 100000
EMBED_DIM = 128
BLK_TC = 4096


def _gather_tc(table, ids):
    n = ids.shape[0]
    grid = (n // BLK_TC,)
    ids3 = ids.reshape(n // BLK_TC, 1, BLK_TC)

    def body(ids_ref, table_ref, out_ref):
        def row(r, carry):
            out_ref[r, :] = table_ref[ids_ref[0, 0, r], :]
            return carry

        lax.fori_loop(0, BLK_TC, row, 0, unroll=8)

    return pl.pallas_call(
        body,
        grid=grid,
        in_specs=[
            pl.BlockSpec((1, 1, BLK_TC), lambda i: (i, 0, 0),
                         memory_space=pltpu.SMEM),
            pl.BlockSpec((VOCAB, EMBED_DIM), lambda i: (0, 0)),
        ],
        out_specs=pl.BlockSpec((BLK_TC, EMBED_DIM), lambda i: (i, 0)),
        out_shape=jax.ShapeDtypeStruct((n, EMBED_DIM), table.dtype),
    )(ids3, table)


def kernel(token_ids, table):
    batch, seq = token_ids.shape
    flat = token_ids.reshape(batch * seq).astype(jnp.int32)
    out = _gather_tc(table, flat)
    return out.reshape(batch, seq, EMBED_DIM)


# speedup vs baseline: 3.3854x; 3.3854x over previous
import functools

import jax
import jax.numpy as jnp
from jax import lax
from jax.experimental import pallas as pl
from jax.experimental.pallas import tpu as pltpu
from jax.experimental.pallas import tpu_sc as plsc

EMBED_DIM = 128
NUM_CORES = 2
NUM_SUBCORES = 16
NUM_WORKERS = NUM_CORES * NUM_SUBCORES
CHUNK = 200
NBUF = 4


def _gather_sc(table, flat_ids):
    num_indices = flat_ids.shape[0]
    per_worker = num_indices // NUM_WORKERS
    nchunks = per_worker // CHUNK
    assert per_worker % CHUNK == 0 and nchunks % NBUF == 0
    mesh = plsc.VectorSubcoreMesh(core_axis_name="c", subcore_axis_name="s")

    @functools.partial(
        pl.kernel,
        mesh=mesh,
        out_type=jax.ShapeDtypeStruct((num_indices, EMBED_DIM), table.dtype),
        scratch_types=[
            pltpu.VMEM((per_worker,), jnp.int32),
            pltpu.VMEM((NBUF, CHUNK, EMBED_DIM), jnp.float32),
            pltpu.SemaphoreType.DMA((NBUF,)),
            pltpu.SemaphoreType.DMA((NBUF,)),
        ],
    )
    def gather_kernel(table_hbm, ids_hbm, out_hbm, idx_v, bufs, gsems, wsems):
        wid = lax.axis_index("s") * NUM_CORES + lax.axis_index("c")
        base = wid * per_worker
        pltpu.sync_copy(ids_hbm.at[pl.ds(base, per_worker)], idx_v)

        def start_gather(c, b):
            pltpu.async_copy(
                table_hbm.at[idx_v.at[pl.ds(c * CHUNK, CHUNK)]],
                bufs.at[b], gsems.at[b])

        def wait_gather(c, b):
            pltpu.make_async_copy(
                table_hbm.at[idx_v.at[pl.ds(c * CHUNK, CHUNK)]],
                bufs.at[b], gsems.at[b]).wait()

        def start_write(c, b):
            pltpu.async_copy(
                bufs.at[b], out_hbm.at[pl.ds(base + c * CHUNK, CHUNK)],
                wsems.at[b])

        def wait_write(c, b):
            pltpu.make_async_copy(
                bufs.at[b], out_hbm.at[pl.ds(base + c * CHUNK, CHUNK)],
                wsems.at[b]).wait()

        for b in range(NBUF):
            start_gather(b, b)

        @pl.loop(0, nchunks, step=NBUF)
        def _(g):
            for b in range(NBUF):
                wait_gather(g + b, b)
                start_write(g + b, b)
            for b in range(NBUF):
                @pl.when(g + b + NBUF < nchunks)
                def _():
                    wait_write(g + b, b)
                    start_gather(g + b + NBUF, b)

        for b in range(NBUF):
            wait_write(nchunks - NBUF + b, b)

    return gather_kernel(table, flat_ids)


def kernel(token_ids, table):
    batch, seq = token_ids.shape
    flat = token_ids.reshape(batch * seq).astype(jnp.int32)
    out = _gather_sc(table, flat)
    return out.reshape(batch, seq, EMBED_DIM)


# speedup vs baseline: 3.4118x; 1.0078x over previous
import functools

import jax
import jax.numpy as jnp
from jax import lax
from jax.experimental import pallas as pl
from jax.experimental.pallas import tpu as pltpu
from jax.experimental.pallas import tpu_sc as plsc

EMBED_DIM = 128
NUM_CORES = 2
NUM_SUBCORES = 16
NUM_WORKERS = NUM_CORES * NUM_SUBCORES
CHUNK = 200
NBUF = 4


def _gather_sc(table, flat_ids):
    num_indices = flat_ids.shape[0]
    per_worker = num_indices // NUM_WORKERS
    nchunks = per_worker // CHUNK
    assert per_worker % CHUNK == 0 and nchunks % NBUF == 0
    mesh = plsc.VectorSubcoreMesh(core_axis_name="c", subcore_axis_name="s")

    @functools.partial(
        pl.kernel,
        mesh=mesh,
        out_type=jax.ShapeDtypeStruct((num_indices, EMBED_DIM), table.dtype),
        scratch_types=[
            pltpu.VMEM((per_worker,), jnp.int32),
            pltpu.VMEM((NBUF, CHUNK, EMBED_DIM), jnp.float32),
            pltpu.SemaphoreType.DMA((NBUF,)),
            pltpu.SemaphoreType.DMA((NBUF,)),
        ],
    )
    def gather_kernel(table_hbm, ids_hbm, out_hbm, idx_v, bufs, gsems, wsems):
        wid = lax.axis_index("s") * NUM_CORES + lax.axis_index("c")
        base = wid * per_worker
        pltpu.sync_copy(ids_hbm.at[pl.ds(base, per_worker)], idx_v)

        def out_row(c):
            return (c * NUM_WORKERS + wid) * CHUNK

        def start_gather(c, b):
            pltpu.async_copy(
                table_hbm.at[idx_v.at[pl.ds(c * CHUNK, CHUNK)]],
                bufs.at[b], gsems.at[b])

        def wait_gather(c, b):
            pltpu.make_async_copy(
                table_hbm.at[idx_v.at[pl.ds(c * CHUNK, CHUNK)]],
                bufs.at[b], gsems.at[b]).wait()

        def start_write(c, b):
            pltpu.async_copy(
                bufs.at[b], out_hbm.at[pl.ds(out_row(c), CHUNK)],
                wsems.at[b])

        def wait_write(c, b):
            pltpu.make_async_copy(
                bufs.at[b], out_hbm.at[pl.ds(out_row(c), CHUNK)],
                wsems.at[b]).wait()

        for b in range(NBUF):
            start_gather(b, b)

        @pl.loop(0, nchunks, step=NBUF)
        def _(g):
            for b in range(NBUF):
                wait_gather(g + b, b)
                start_write(g + b, b)
            for b in range(NBUF):
                @pl.when(g + b + NBUF < nchunks)
                def _():
                    wait_write(g + b, b)
                    start_gather(g + b + NBUF, b)

        for b in range(NBUF):
            wait_write(nchunks - NBUF + b, b)

    return gather_kernel(table, flat_ids)


def kernel(token_ids, table):
    batch, seq = token_ids.shape
    n = batch * seq
    nchunks = n // (NUM_WORKERS * CHUNK)
    flat = token_ids.reshape(n).astype(jnp.int32)
    perm = flat.reshape(nchunks, NUM_WORKERS, CHUNK).transpose(1, 0, 2)
    out = _gather_sc(table, perm.reshape(n))
    return out.reshape(batch, seq, EMBED_DIM)
